# baseline (speedup 1.0000x reference)
import jax
import jax.numpy as jnp
from jax.experimental import pallas as pl
from jax.experimental.pallas import tpu as pltpu

_N = 10000
_IN_F = 128
_MID = 32
_OUT_F = 128
_BM = 400
_NBUF = 3
_OBUF = 2
_NSTEPS = _N // _BM


def _gcn_kernel(x_ref, w1_ref, w2_ref, adj_ref, out_ref,
                abuf, obuf, hid, in_sems, out_sems):
    for b in range(_NBUF):
        pltpu.make_async_copy(
            adj_ref.at[pl.ds(b * _BM, _BM), :], abuf.at[b], in_sems.at[b]
        ).start()

    hid[...] = jnp.dot(
        x_ref[...].astype(jnp.bfloat16),
        w1_ref[...].astype(jnp.bfloat16),
        preferred_element_type=jnp.float32,
    ).astype(jnp.bfloat16)

    def step(i, carry):
        slot = jax.lax.rem(i, _NBUF)
        oslot = jax.lax.rem(i, _OBUF)
        row = i * _BM
        pltpu.make_async_copy(
            adj_ref.at[pl.ds(row, _BM), :], abuf.at[slot], in_sems.at[slot]
        ).wait()

        @pl.when(i >= _OBUF)
        def _():
            pltpu.make_async_copy(
                obuf.at[oslot],
                out_ref.at[pl.ds((i - _OBUF) * _BM, _BM), :],
                out_sems.at[oslot],
            ).wait()

        t = jnp.dot(
            abuf[slot].astype(jnp.bfloat16),
            hid[...],
            preferred_element_type=jnp.float32,
        )
        obuf[oslot] = jnp.dot(
            t.astype(jnp.bfloat16),
            w2_ref[...].astype(jnp.bfloat16),
            preferred_element_type=jnp.float32,
        )
        pltpu.make_async_copy(
            obuf.at[oslot], out_ref.at[pl.ds(row, _BM), :], out_sems.at[oslot]
        ).start()

        nxt = i + _NBUF
        @pl.when(nxt < _NSTEPS)
        def _():
            pltpu.make_async_copy(
                adj_ref.at[pl.ds(nxt * _BM, _BM), :], abuf.at[slot], in_sems.at[slot]
            ).start()
        return carry

    jax.lax.fori_loop(0, _NSTEPS, step, 0)

    for k in range(_NSTEPS - _OBUF, _NSTEPS):
        pltpu.make_async_copy(
            obuf.at[k % _OBUF], out_ref.at[pl.ds(k * _BM, _BM), :],
            out_sems.at[k % _OBUF],
        ).wait()


def kernel(input, adj, weight1, weight2):
    return pl.pallas_call(
        _gcn_kernel,
        in_specs=[
            pl.BlockSpec(memory_space=pltpu.MemorySpace.VMEM),
            pl.BlockSpec(memory_space=pltpu.MemorySpace.VMEM),
            pl.BlockSpec(memory_space=pltpu.MemorySpace.VMEM),
            pl.BlockSpec(memory_space=pltpu.MemorySpace.HBM),
        ],
        out_specs=pl.BlockSpec(memory_space=pltpu.MemorySpace.HBM),
        out_shape=jax.ShapeDtypeStruct((_N, _OUT_F), jnp.float32),
        scratch_shapes=[
            pltpu.VMEM((_NBUF, _BM, _N), jnp.float32),
            pltpu.VMEM((_OBUF, _BM, _OUT_F), jnp.float32),
            pltpu.VMEM((_N, _MID), jnp.bfloat16),
            pltpu.SemaphoreType.DMA((_NBUF,)),
            pltpu.SemaphoreType.DMA((_OBUF,)),
        ],
    )(input, weight1, weight2, adj)


# speedup vs baseline: 1.0058x; 1.0058x over previous
import jax
import jax.numpy as jnp
from jax.experimental import pallas as pl
from jax.experimental.pallas import tpu as pltpu

_N = 10000
_IN_F = 128
_MID = 32
_OUT_F = 128
_BM = 200
_NBUF = 5
_NSTEPS = _N // _BM
_NOUTER = _NSTEPS // _NBUF


def _gcn_kernel(x_ref, w1_ref, w2_ref, adj_ref, out_ref,
                b0, b1, b2, b3, b4, ovmem, hid, in_sems, out_sem):
    bufs = (b0, b1, b2, b3, b4)

    for b in range(_NBUF):
        pltpu.make_async_copy(
            adj_ref.at[pl.ds(b * _BM, _BM), :], bufs[b], in_sems.at[b]
        ).start()

    hid[...] = jnp.dot(
        x_ref[...].astype(jnp.bfloat16),
        w1_ref[...].astype(jnp.bfloat16),
        preferred_element_type=jnp.float32,
    ).astype(jnp.bfloat16)

    def outer(o, carry):
        base = o * _NBUF
        for b in range(_NBUF):
            i = base + b
            row = i * _BM
            pltpu.make_async_copy(
                adj_ref.at[pl.ds(row, _BM), :], bufs[b], in_sems.at[b]
            ).wait()
            t = jnp.dot(
                bufs[b][...].astype(jnp.bfloat16),
                hid[...],
                preferred_element_type=jnp.float32,
            )
            ovmem[pl.ds(row, _BM), :] = jnp.dot(
                t.astype(jnp.bfloat16),
                w2_ref[...].astype(jnp.bfloat16),
                preferred_element_type=jnp.float32,
            )
            pltpu.make_async_copy(
                ovmem.at[pl.ds(row, _BM), :], out_ref.at[pl.ds(row, _BM), :],
                out_sem,
            ).start()
            nxt = i + _NBUF
            @pl.when(nxt < _NSTEPS)
            def _():
                pltpu.make_async_copy(
                    adj_ref.at[pl.ds(nxt * _BM, _BM), :], bufs[b], in_sems.at[b]
                ).start()
        return carry

    jax.lax.fori_loop(0, _NOUTER, outer, 0)

    def drain(i, carry):
        row = i * _BM
        pltpu.make_async_copy(
            ovmem.at[pl.ds(row, _BM), :], out_ref.at[pl.ds(row, _BM), :], out_sem
        ).wait()
        return carry

    jax.lax.fori_loop(0, _NSTEPS, drain, 0)


def kernel(input, adj, weight1, weight2):
    return pl.pallas_call(
        _gcn_kernel,
        in_specs=[
            pl.BlockSpec(memory_space=pltpu.MemorySpace.VMEM),
            pl.BlockSpec(memory_space=pltpu.MemorySpace.VMEM),
            pl.BlockSpec(memory_space=pltpu.MemorySpace.VMEM),
            pl.BlockSpec(memory_space=pltpu.MemorySpace.HBM),
        ],
        out_specs=pl.BlockSpec(memory_space=pltpu.MemorySpace.HBM),
        out_shape=jax.ShapeDtypeStruct((_N, _OUT_F), jnp.float32),
        scratch_shapes=[
            pltpu.VMEM((_BM, _N), jnp.float32),
            pltpu.VMEM((_BM, _N), jnp.float32),
            pltpu.VMEM((_BM, _N), jnp.float32),
            pltpu.VMEM((_BM, _N), jnp.float32),
            pltpu.VMEM((_BM, _N), jnp.float32),
            pltpu.VMEM((_N, _OUT_F), jnp.float32),
            pltpu.VMEM((_N, _MID), jnp.bfloat16),
            pltpu.SemaphoreType.DMA((_NBUF,)),
            pltpu.SemaphoreType.DMA,
        ],
    )(input, weight1, weight2, adj)


# speedup vs baseline: 1.0247x; 1.0187x over previous
import jax
import jax.numpy as jnp
from jax.experimental import pallas as pl
from jax.experimental.pallas import tpu as pltpu

_N = 10000
_IN_F = 128
_MID = 32
_OUT_F = 128
_BM = 400


def _gcn_kernel(x_ref, w1_ref, adj_ref, w2_ref, out_ref, hid_ref):
    @pl.when(pl.program_id(0) == 0)
    def _():
        h = jnp.dot(
            x_ref[...].astype(jnp.bfloat16),
            w1_ref[...].astype(jnp.bfloat16),
            preferred_element_type=jnp.float32,
        )
        hid_ref[...] = h.astype(jnp.bfloat16)

    t = jnp.dot(
        adj_ref[...].astype(jnp.bfloat16),
        hid_ref[...],
        preferred_element_type=jnp.float32,
    )
    out_ref[...] = jnp.dot(
        t.astype(jnp.bfloat16),
        w2_ref[...].astype(jnp.bfloat16),
        preferred_element_type=jnp.float32,
    )


def kernel(input, adj, weight1, weight2):
    grid = (_N // _BM,)
    return pl.pallas_call(
        _gcn_kernel,
        grid=grid,
        in_specs=[
            pl.BlockSpec((_N, _IN_F), lambda i: (0, 0)),
            pl.BlockSpec((_IN_F, _MID), lambda i: (0, 0)),
            pl.BlockSpec((_BM, _N), lambda i: (i, 0)),
            pl.BlockSpec((_MID, _OUT_F), lambda i: (0, 0)),
        ],
        out_specs=pl.BlockSpec((_BM, _OUT_F), lambda i: (i, 0)),
        out_shape=jax.ShapeDtypeStruct((_N, _OUT_F), jnp.float32),
        scratch_shapes=[pltpu.VMEM((_N, _MID), jnp.bfloat16)],
        compiler_params=pltpu.CompilerParams(
            dimension_semantics=("arbitrary",),
        ),
    )(input, weight1, adj, weight2)
